# baseline (speedup 1.0000x reference)
import functools
import math

import jax
import jax.numpy as jnp
from jax import lax
from jax.experimental import pallas as pl
from jax.experimental.pallas import tpu as pltpu
from jax.experimental.pallas import tpu_sc as plsc

D_H = 64
BBLK = 128
NBUF = 4
NUM_CORES = 2
NUM_SUBCORES = 16
NW = NUM_CORES * NUM_SUBCORES


def _emb_body(seq_len, n_batch, src_hbm, pe_hbm, table_hbm, out_hbm,
              idxs, rows_v, obuf, pe_v,
              gsem0, gsem1, gsem2, gsem3, ssem0, ssem1):
    gsem = (gsem0, gsem1, gsem2, gsem3)
    ssem = (ssem0, ssem1)
    wid = lax.axis_index("s") * NUM_CORES + lax.axis_index("c")
    b0 = wid * BBLK

    pltpu.sync_copy(src_hbm.at[:, pl.ds(b0, BBLK)], idxs)
    pltpu.sync_copy(pe_hbm, pe_v)

    def fire_gather(l, b):
        pltpu.async_copy(table_hbm.at[idxs.at[l]], rows_v.at[b], gsem[b])

    def wait_gather(b):
        pltpu.make_async_copy(table_hbm.at[idxs.at[0]], rows_v.at[b],
                              gsem[b]).wait()

    def fire_store(l, b):
        pltpu.async_copy(obuf.at[b], out_hbm.at[l, :, pl.ds(b0, BBLK)],
                         ssem[b])

    def wait_store(b):
        pltpu.make_async_copy(obuf.at[b], out_hbm.at[0, :, pl.ds(b0, BBLK)],
                              ssem[b]).wait()

    def compute(l, b, ob):
        @pl.loop(0, BBLK // 2, unroll=4)
        def _(j2):
            for jj in range(2):
                j = j2 * 2 + jj
                jb = jnp.broadcast_to(j, (16,))
                for k in range(D_H // 16):
                    cidx = k * 16 + lax.iota(jnp.int32, 16)
                    v = rows_v[b, j, pl.ds(k * 16, 16)]
                    p = pe_v[l, pl.ds(k * 16, 16)]
                    plsc.store_scatter(obuf.at[ob], [cidx, jb], v * 8.0 + p)

    def l_step(l, b, ob):
        @pl.when(l + (NBUF - 1) < seq_len)
        def _():
            fire_gather(l + (NBUF - 1), (b + NBUF - 1) % NBUF)

        wait_gather(b)

        @pl.when(l >= 2)
        def _():
            wait_store(ob)

        compute(l, b, ob)
        fire_store(l, ob)

    for l in range(NBUF - 1):
        fire_gather(l, l)

    @pl.loop(0, seq_len // NBUF)
    def _(l4):
        for u in range(NBUF):
            l_step(l4 * NBUF + u, u, u % 2)

    wait_store(0)
    wait_store(1)


def _build_sc_call(n_batch, seq_len):
    mesh = plsc.VectorSubcoreMesh(core_axis_name="c", subcore_axis_name="s")
    return functools.partial(
        pl.kernel,
        out_type=jax.ShapeDtypeStruct((seq_len, D_H, n_batch), jnp.float32),
        mesh=mesh,
        scratch_types=[
            pltpu.VMEM((seq_len, BBLK), jnp.int32),
            pltpu.VMEM((NBUF, BBLK, D_H), jnp.float32),
            pltpu.VMEM((2, D_H, BBLK), jnp.float32),
            pltpu.VMEM((seq_len, D_H), jnp.float32),
            pltpu.SemaphoreType.DMA,
            pltpu.SemaphoreType.DMA,
            pltpu.SemaphoreType.DMA,
            pltpu.SemaphoreType.DMA,
            pltpu.SemaphoreType.DMA,
            pltpu.SemaphoreType.DMA,
        ],
        compiler_params=pltpu.CompilerParams(use_tc_tiling_on_sc=False,
                                             needs_layout_passes=False),
    )(functools.partial(_emb_body, seq_len, n_batch))


def kernel(src, table, pe):
    b, l = src.shape
    assert b == NW * BBLK and l % (2 * NBUF) == 0
    srcT = src.T * 2
    proj = jnp.concatenate(
        [jnp.eye(D_H, dtype=table.dtype),
         jnp.zeros((D_H, D_H), table.dtype)], axis=1)
    tpad = jnp.matmul(table, proj,
                      precision=jax.lax.Precision.HIGHEST
                      ).reshape(2 * table.shape[0], D_H)
    pe_seq = pe[0, :l, :]
    out2 = _build_sc_call(b, l)(srcT, pe_seq, tpad)
    return jnp.transpose(out2, (2, 0, 1))


# speedup vs baseline: 1.5925x; 1.5925x over previous
import functools
import math

import jax
import jax.numpy as jnp
from jax import lax
from jax.experimental import pallas as pl
from jax.experimental.pallas import tpu as pltpu
from jax.experimental.pallas import tpu_sc as plsc

D_H = 64
BBLK = 128
NBUF = 4
NUM_CORES = 2
NUM_SUBCORES = 16
NW = NUM_CORES * NUM_SUBCORES


def _emb_body(seq_len, n_batch, src_hbm, pe_hbm, table_hbm, out_hbm,
              idxs, rows_v, obuf, pe_v,
              gsem0, gsem1, gsem2, gsem3, ssem0, ssem1):
    gsem = (gsem0, gsem1, gsem2, gsem3)
    ssem = (ssem0, ssem1)
    wid = lax.axis_index("s") * NUM_CORES + lax.axis_index("c")
    b0 = wid * BBLK

    pltpu.sync_copy(src_hbm.at[:, pl.ds(b0, BBLK)], idxs)
    pltpu.sync_copy(pe_hbm, pe_v)

    def fire_gather(l, b):
        pltpu.async_copy(table_hbm.at[idxs.at[l]], rows_v.at[b], gsem[b])

    def wait_gather(b):
        pltpu.make_async_copy(table_hbm.at[idxs.at[0]], rows_v.at[b],
                              gsem[b]).wait()

    def fire_store(l, b):
        pltpu.async_copy(obuf.at[b, :, pl.ds(0, BBLK)],
                         out_hbm.at[l, :, pl.ds(b0, BBLK)], ssem[b])

    def wait_store(b):
        pltpu.make_async_copy(obuf.at[b, :, pl.ds(0, BBLK)],
                              out_hbm.at[0, :, pl.ds(b0, BBLK)],
                              ssem[b]).wait()

    def compute(l, b, ob):
        @pl.loop(0, BBLK // 2, unroll=4)
        def _(j2):
            for jj in range(2):
                j = j2 * 2 + jj
                jb = jnp.broadcast_to(j, (16,))
                for k in range(D_H // 16):
                    cidx = k * 16 + lax.iota(jnp.int32, 16)
                    v = rows_v[b, j, pl.ds(k * 16, 16)]
                    p = pe_v[l, pl.ds(k * 16, 16)]
                    plsc.store_scatter(obuf.at[ob], [cidx, jb], v * 8.0 + p)

    def l_step(l, b, ob):
        @pl.when(l + (NBUF - 1) < seq_len)
        def _():
            fire_gather(l + (NBUF - 1), (b + NBUF - 1) % NBUF)

        wait_gather(b)

        @pl.when(l >= 2)
        def _():
            wait_store(ob)

        compute(l, b, ob)
        fire_store(l, ob)

    for l in range(NBUF - 1):
        fire_gather(l, l)

    @pl.loop(0, seq_len // NBUF)
    def _(l4):
        for u in range(NBUF):
            l_step(l4 * NBUF + u, u, u % 2)

    wait_store(0)
    wait_store(1)


def _build_sc_call(n_batch, seq_len):
    mesh = plsc.VectorSubcoreMesh(core_axis_name="c", subcore_axis_name="s")
    return functools.partial(
        pl.kernel,
        out_type=jax.ShapeDtypeStruct((seq_len, D_H, n_batch), jnp.float32),
        mesh=mesh,
        scratch_types=[
            pltpu.VMEM((seq_len, BBLK), jnp.int32),
            pltpu.VMEM((NBUF, BBLK, D_H), jnp.float32),
            pltpu.VMEM((2, D_H, BBLK + 1), jnp.float32),
            pltpu.VMEM((seq_len, D_H), jnp.float32),
            pltpu.SemaphoreType.DMA,
            pltpu.SemaphoreType.DMA,
            pltpu.SemaphoreType.DMA,
            pltpu.SemaphoreType.DMA,
            pltpu.SemaphoreType.DMA,
            pltpu.SemaphoreType.DMA,
        ],
        compiler_params=pltpu.CompilerParams(use_tc_tiling_on_sc=False,
                                             needs_layout_passes=False),
    )(functools.partial(_emb_body, seq_len, n_batch))


def kernel(src, table, pe):
    b, l = src.shape
    assert b == NW * BBLK and l % (2 * NBUF) == 0
    srcT = src.T * 2
    tpad = jnp.concatenate(
        [table, jnp.zeros((table.shape[0], D_H), table.dtype)],
        axis=1).reshape(2 * table.shape[0], D_H)
    pe_seq = pe[0, :l, :]
    out2 = _build_sc_call(b, l)(srcT, pe_seq, tpad)
    return jnp.transpose(out2, (2, 0, 1))


# speedup vs baseline: 1.9141x; 1.2019x over previous
import functools
import math

import jax
import jax.numpy as jnp
from jax import lax
from jax.experimental import pallas as pl
from jax.experimental.pallas import tpu as pltpu
from jax.experimental.pallas import tpu_sc as plsc

D_H = 64
BBLK = 128
NBUF = 8
NUM_CORES = 2
NUM_SUBCORES = 16
NW = NUM_CORES * NUM_SUBCORES


def _emb_body(seq_len, n_batch, src_hbm, pe_hbm, table_hbm, out_hbm,
              idxs, rows_v, obuf, pe_v, gsems, ssems):
    wid = lax.axis_index("s") * NUM_CORES + lax.axis_index("c")
    b0 = wid * BBLK

    pltpu.sync_copy(src_hbm.at[:, pl.ds(b0, BBLK)], idxs)
    pltpu.sync_copy(pe_hbm, pe_v)

    def fire_gather(l, b):
        pltpu.async_copy(table_hbm.at[idxs.at[l]], rows_v.at[b], gsems[b])

    def wait_gather(b):
        pltpu.make_async_copy(table_hbm.at[idxs.at[0]], rows_v.at[b],
                              gsems[b]).wait()

    def fire_store(l, ob):
        for ct in range(D_H // 8):
            pltpu.async_copy(obuf.at[ob, ct, :, pl.ds(0, BBLK)],
                             out_hbm.at[l, ct, wid], ssems[ob])

    def wait_store(ob):
        for ct in range(D_H // 8):
            pltpu.make_async_copy(obuf.at[ob, ct, :, pl.ds(0, BBLK)],
                                  out_hbm.at[0, ct, wid], ssems[ob]).wait()

    def compute(l, b, ob):
        ci = lax.iota(jnp.int32, 16)
        pv = [pe_v[l, pl.ds(k * 16, 16)] for k in range(D_H // 16)]
        cts = [(ci + k * 16) >> 3 for k in range(D_H // 16)]
        css = [(ci + k * 16) & 7 for k in range(D_H // 16)]

        @pl.loop(0, BBLK // 2, unroll=4)
        def _(j2):
            for jj in range(2):
                j = j2 * 2 + jj
                jb = jnp.broadcast_to(j, (16,))
                for k in range(D_H // 16):
                    v = rows_v[b, j, pl.ds(k * 16, 16)]
                    plsc.store_scatter(obuf.at[ob], [cts[k], css[k], jb],
                                       v * 8.0 + pv[k])

    def l_step(l, b, ob):
        @pl.when(l + (NBUF - 1) < seq_len)
        def _():
            fire_gather(l + (NBUF - 1), (b + NBUF - 1) % NBUF)

        wait_gather(b)

        @pl.when(l >= 2)
        def _():
            wait_store(ob)

        compute(l, b, ob)
        fire_store(l, ob)

    for l in range(NBUF - 1):
        fire_gather(l, l)

    @pl.loop(0, seq_len // NBUF)
    def _(lg):
        for u in range(NBUF):
            l_step(lg * NBUF + u, u, u % 2)

    wait_store(0)
    wait_store(1)


def _build_sc_call(n_batch, seq_len):
    mesh = plsc.VectorSubcoreMesh(core_axis_name="c", subcore_axis_name="s")
    nbt = n_batch // BBLK
    return functools.partial(
        pl.kernel,
        out_type=jax.ShapeDtypeStruct((seq_len, D_H // 8, nbt, 8, BBLK),
                                      jnp.float32),
        mesh=mesh,
        scratch_types=[
            pltpu.VMEM((seq_len, BBLK), jnp.int32),
            pltpu.VMEM((NBUF, BBLK, D_H), jnp.float32),
            pltpu.VMEM((2, D_H // 8, 8, BBLK + 1), jnp.float32),
            pltpu.VMEM((seq_len, D_H), jnp.float32),
            [pltpu.SemaphoreType.DMA] * NBUF,
            [pltpu.SemaphoreType.DMA] * 2,
        ],
        compiler_params=pltpu.CompilerParams(use_tc_tiling_on_sc=False,
                                             needs_layout_passes=False),
    )(functools.partial(_emb_body, seq_len, n_batch))


def kernel(src, table, pe):
    b, l = src.shape
    assert b == NW * BBLK and l % NBUF == 0
    srcT = src.T * 2
    tpad = jnp.concatenate(
        [table, jnp.zeros((table.shape[0], D_H), table.dtype)],
        axis=1).reshape(2 * table.shape[0], D_H)
    pe_seq = pe[0, :l, :]
    out5 = _build_sc_call(b, l)(srcT, pe_seq, tpad)
    return out5.transpose(2, 4, 0, 1, 3).reshape(b, l, D_H)
